# baseline (speedup 1.0000x reference)
import functools

import jax
import jax.numpy as jnp
from jax import lax
from jax.experimental import pallas as pl
from jax.experimental.pallas import tpu as pltpu
from jax.experimental.pallas import tpu_sc as plsc

D = 512
L = 16


def _sc_add_emb(x2d, idx, table):
    N = x2d.shape[0]
    info = plsc.get_sparse_core_info()
    NC, NS = info.num_cores, info.num_subcores
    NW = NC * NS
    C = 64
    n_w = N // NW
    n_chunks = n_w // C
    mesh = plsc.VectorSubcoreMesh(core_axis_name="c", subcore_axis_name="s")

    @functools.partial(
        pl.kernel,
        mesh=mesh,
        out_type=jax.ShapeDtypeStruct((N, D), jnp.float32),
        scratch_types=[
            pltpu.VMEM((C,), jnp.int32),
            pltpu.VMEM((C, D), jnp.float32),
            pltpu.VMEM((C, D), jnp.float32),
            pltpu.SemaphoreType.DMA,
        ],
    )
    def k(x_hbm, idx_hbm, tab_hbm, out_hbm, idx_v, x_v, rows_v, sem):
        wid = lax.axis_index("s") * NC + lax.axis_index("c")
        wbase = wid * n_w

        def chunk(g, carry):
            base = wbase + g * C
            pltpu.sync_copy(idx_hbm.at[pl.ds(base, C)], idx_v)
            pltpu.sync_copy(x_hbm.at[pl.ds(base, C)], x_v)
            pltpu.async_copy(tab_hbm.at[idx_v], rows_v, sem).wait()

            def row(c, carry2):
                for j in range(D // L):
                    plsc.addupdate(
                        x_v.at[c, pl.ds(j * L, L)],
                        rows_v[c, pl.ds(j * L, L)],
                    )
                return carry2

            lax.fori_loop(0, C, row, 0)
            pltpu.sync_copy(x_v, out_hbm.at[pl.ds(base, C)])
            return carry

        lax.fori_loop(0, n_chunks, chunk, 0)

    return k(x2d, idx, table)


def kernel(x, instruction_types, type_emb_weight):
    B, S, d = x.shape
    x2d = x.reshape(B * S, d)
    idx = instruction_types.reshape(-1).astype(jnp.int32)
    out = _sc_add_emb(x2d, idx, type_emb_weight)
    return out.reshape(B, S, d)


# speedup vs baseline: 1.1942x; 1.1942x over previous
import functools

import jax
import jax.numpy as jnp
from jax import lax
from jax.experimental import pallas as pl
from jax.experimental.pallas import tpu as pltpu
from jax.experimental.pallas import tpu_sc as plsc

D = 512
L = 16
C = 32
NBUF = 3


def _sc_add_emb(x2d, idx, table):
    N = x2d.shape[0]
    info = plsc.get_sparse_core_info()
    NC, NS = info.num_cores, info.num_subcores
    NW = NC * NS
    n_w = N // NW
    n_chunks = n_w // C
    mesh = plsc.VectorSubcoreMesh(core_axis_name="c", subcore_axis_name="s")

    @functools.partial(
        pl.kernel,
        mesh=mesh,
        out_type=jax.ShapeDtypeStruct((N, D), jnp.float32),
        scratch_types=[
            pltpu.VMEM((n_w,), jnp.int32),
            pltpu.VMEM((NBUF, C, D), jnp.float32),
            pltpu.VMEM((NBUF, C, D), jnp.float32),
            pltpu.SemaphoreType.DMA((NBUF,)),
            pltpu.SemaphoreType.DMA((NBUF,)),
            pltpu.SemaphoreType.DMA((NBUF,)),
        ],
    )
    def k(x_hbm, idx_hbm, tab_hbm, out_hbm,
          idx_all, x_v, rows_v, sem_x, sem_g, sem_o):
        wid = lax.axis_index("s") * NC + lax.axis_index("c")
        wbase = wid * n_w
        pltpu.sync_copy(idx_hbm.at[pl.ds(wbase, n_w)], idx_all)

        def in_copies(g, b):
            base = pl.multiple_of(wbase + g * C, C)
            ibase = pl.multiple_of(g * C, C)
            return (
                pltpu.make_async_copy(
                    x_hbm.at[pl.ds(base, C)], x_v.at[b], sem_x.at[b]),
                pltpu.make_async_copy(
                    tab_hbm.at[idx_all.at[pl.ds(ibase, C)]],
                    rows_v.at[b], sem_g.at[b]),
            )

        def out_copy(g, b):
            base = pl.multiple_of(wbase + g * C, C)
            return pltpu.make_async_copy(
                x_v.at[b], out_hbm.at[pl.ds(base, C)], sem_o.at[b])

        def issue_in(g):
            b = lax.rem(g, NBUF)
            for cp in in_copies(g, b):
                cp.start()

        for g0 in range(NBUF - 1):
            issue_in(g0)

        def body(g, carry):
            b = lax.rem(g, NBUF)
            for cp in in_copies(g, b):
                cp.wait()

            def row(c, carry2):
                for j in range(D // L):
                    plsc.addupdate(
                        x_v.at[b, c, pl.ds(j * L, L)],
                        rows_v[b, c, pl.ds(j * L, L)],
                    )
                return carry2

            lax.fori_loop(0, C, row, 0)
            out_copy(g, b).start()

            g2 = g + NBUF - 1
            b2 = lax.rem(g2, NBUF)

            @pl.when(jnp.logical_and(g >= 1, g2 < n_chunks))
            def _():
                out_copy(g - 1, b2).wait()

            @pl.when(g2 < n_chunks)
            def _():
                issue_in(g2)

            return carry

        lax.fori_loop(0, n_chunks, body, 0)

        for gd in range(n_chunks - NBUF, n_chunks):
            out_copy(gd, gd % NBUF).wait()

    return k(x2d, idx, table)


def kernel(x, instruction_types, type_emb_weight):
    B, S, d = x.shape
    x2d = x.reshape(B * S, d)
    idx = instruction_types.reshape(-1).astype(jnp.int32)
    out = _sc_add_emb(x2d, idx, type_emb_weight)
    return out.reshape(B, S, d)
